# baseline (speedup 1.0000x reference)
import functools

import jax
import jax.numpy as jnp
from jax import lax
from jax.experimental import pallas as pl
from jax.experimental.pallas import tpu as pltpu
from jax.experimental.pallas import tpu_sc as plsc

_NUM_ENT = 100000
_DIM = 128
_NUM_REL2 = 400
_BATCH = 16384

_NC = 2
_NS = 16
_NW = _NC * _NS
_BPW = _BATCH // _NW
_CHUNK = 128
_NCHUNK = _BPW // _CHUNK
_IDX_ROWS_PER_W = _BPW // _CHUNK


def _gather_body(emb_hbm, reltab_hbm, sub_hbm, rel_hbm,
                 sub_out, rel_out,
                 sub_idx_v, rel_idx_v, rows_a, rows_b, sem_a, sem_b):
    c = lax.axis_index("c")
    s = lax.axis_index("s")
    wid = s * _NC + c
    base = wid * _BPW
    irow = wid * _IDX_ROWS_PER_W

    pltpu.sync_copy(sub_hbm.at[pl.ds(irow, _NCHUNK)], sub_idx_v)
    pltpu.sync_copy(rel_hbm.at[pl.ds(irow, _NCHUNK)], rel_idx_v)

    tasks = [(sub_idx_v, emb_hbm, sub_out, j) for j in range(_NCHUNK)] + \
            [(rel_idx_v, reltab_hbm, rel_out, j) for j in range(_NCHUNK)]
    bufs = [(rows_a, sem_a), (rows_b, sem_b)]

    idx0, tab0, _, j0 = tasks[0]
    cp0 = pltpu.async_copy(tab0.at[idx0.at[j0]], bufs[0][0], bufs[0][1])
    pending = [cp0]
    for k, (idx, tab, out, j) in enumerate(tasks):
        buf, sem = bufs[k % 2]
        if k + 1 < len(tasks):
            nidx, ntab, _, nj = tasks[k + 1]
            nbuf, nsem = bufs[(k + 1) % 2]
            pending.append(
                pltpu.async_copy(ntab.at[nidx.at[nj]], nbuf, nsem))
        pending[k].wait()
        pltpu.sync_copy(buf, out.at[pl.ds(base + j * _CHUNK, _CHUNK)])


@functools.partial(
    pl.kernel,
    out_type=(
        jax.ShapeDtypeStruct((_BATCH, _DIM), jnp.float32),
        jax.ShapeDtypeStruct((_BATCH, _DIM), jnp.float32),
    ),
    mesh=plsc.VectorSubcoreMesh(core_axis_name="c", subcore_axis_name="s"),
    scratch_types=(
        pltpu.VMEM((_NCHUNK, _CHUNK), jnp.int32),
        pltpu.VMEM((_NCHUNK, _CHUNK), jnp.int32),
        pltpu.VMEM((_CHUNK, _DIM), jnp.float32),
        pltpu.VMEM((_CHUNK, _DIM), jnp.float32),
        pltpu.SemaphoreType.DMA,
        pltpu.SemaphoreType.DMA,
    ),
)
def _sc_gathers(emb_hbm, reltab_hbm, sub_hbm, rel_hbm, sub_out, rel_out,
                sub_idx_v, rel_idx_v, rows_a, rows_b, sem_a, sem_b):
    _gather_body(emb_hbm, reltab_hbm, sub_hbm, rel_hbm, sub_out, rel_out,
                 sub_idx_v, rel_idx_v, rows_a, rows_b, sem_a, sem_b)


def kernel(init_embed, init_rel, edge_index, edge_type, sub, rel):
    sub2 = sub.astype(jnp.int32).reshape(_BATCH // _CHUNK, _CHUNK)
    rel2 = rel.astype(jnp.int32).reshape(_BATCH // _CHUNK, _CHUNK)
    sub_emb, rel_emb = _sc_gathers(init_embed, init_rel, sub2, rel2)
    return (sub_emb, rel_emb, init_embed)


# speedup vs baseline: 1.0780x; 1.0780x over previous
import functools

import jax
import jax.numpy as jnp
from jax import lax
from jax.experimental import pallas as pl
from jax.experimental.pallas import tpu as pltpu
from jax.experimental.pallas import tpu_sc as plsc

_NUM_ENT = 100000
_DIM = 128
_NUM_REL2 = 400
_BATCH = 16384

_NC = 2
_NS = 16
_NW = _NC * _NS
_BPW = _BATCH // _NW
_CHUNK = 128
_NCHUNK = _BPW // _CHUNK
_IDX_ROWS_PER_W = _BPW // _CHUNK


def _gather_body(emb_hbm, reltab_hbm, sub_hbm, rel_hbm,
                 sub_out, rel_out,
                 sub_idx_v, rel_idx_v, rows_a, rows_b, sem_a, sem_b):
    c = lax.axis_index("c")
    s = lax.axis_index("s")
    wid = s * _NC + c
    base = wid * _BPW
    irow = wid * _IDX_ROWS_PER_W

    pltpu.sync_copy(sub_hbm.at[pl.ds(irow, _NCHUNK)], sub_idx_v)
    pltpu.sync_copy(rel_hbm.at[pl.ds(irow, _NCHUNK)], rel_idx_v)

    tasks = [(sub_idx_v, emb_hbm, sub_out, j) for j in range(_NCHUNK)] + \
            [(rel_idx_v, reltab_hbm, rel_out, j) for j in range(_NCHUNK)]
    bufs = [(rows_a, sem_a), (rows_b, sem_b)]

    idx0, tab0, _, j0 = tasks[0]
    cp0 = pltpu.async_copy(tab0.at[idx0.at[j0]], bufs[0][0], bufs[0][1])
    pending = [cp0]
    for k, (idx, tab, out, j) in enumerate(tasks):
        buf, sem = bufs[k % 2]
        if k + 1 < len(tasks):
            nidx, ntab, _, nj = tasks[k + 1]
            nbuf, nsem = bufs[(k + 1) % 2]
            pending.append(
                pltpu.async_copy(ntab.at[nidx.at[nj]], nbuf, nsem))
        pending[k].wait()
        pltpu.sync_copy(buf, out.at[pl.ds(base + j * _CHUNK, _CHUNK)])


@functools.partial(
    pl.kernel,
    out_type=(
        jax.ShapeDtypeStruct((_BATCH, _DIM), jnp.float32),
        jax.ShapeDtypeStruct((_BATCH, _DIM), jnp.float32),
    ),
    mesh=plsc.VectorSubcoreMesh(core_axis_name="c", subcore_axis_name="s"),
    scratch_types=(
        pltpu.VMEM((_NCHUNK, _CHUNK), jnp.int32),
        pltpu.VMEM((_NCHUNK, _CHUNK), jnp.int32),
        pltpu.VMEM((_CHUNK, _DIM), jnp.float32),
        pltpu.VMEM((_CHUNK, _DIM), jnp.float32),
        pltpu.SemaphoreType.DMA,
        pltpu.SemaphoreType.DMA,
    ),
)
def _sc_gathers(emb_hbm, reltab_hbm, sub_hbm, rel_hbm, sub_out, rel_out,
                sub_idx_v, rel_idx_v, rows_a, rows_b, sem_a, sem_b):
    _gather_body(emb_hbm, reltab_hbm, sub_hbm, rel_hbm, sub_out, rel_out,
                 sub_idx_v, rel_idx_v, rows_a, rows_b, sem_a, sem_b)


_COPY_ROWS = 5000


def _copy_body(x_ref, o_ref):
    o_ref[...] = x_ref[...]


_tc_copy = pl.pallas_call(
    _copy_body,
    out_shape=jax.ShapeDtypeStruct((_NUM_ENT, _DIM), jnp.float32),
    grid=(_NUM_ENT // _COPY_ROWS,),
    in_specs=[pl.BlockSpec((_COPY_ROWS, _DIM), lambda i: (i, 0))],
    out_specs=pl.BlockSpec((_COPY_ROWS, _DIM), lambda i: (i, 0)),
)


def kernel(init_embed, init_rel, edge_index, edge_type, sub, rel):
    sub2 = sub.astype(jnp.int32).reshape(_BATCH // _CHUNK, _CHUNK)
    rel2 = rel.astype(jnp.int32).reshape(_BATCH // _CHUNK, _CHUNK)
    sub_emb, rel_emb = _sc_gathers(init_embed, init_rel, sub2, rel2)
    x_out = _tc_copy(init_embed)
    return (sub_emb, rel_emb, x_out)


# speedup vs baseline: 1.1032x; 1.0234x over previous
import functools

import jax
import jax.numpy as jnp
from jax import lax
from jax.experimental import pallas as pl
from jax.experimental.pallas import tpu as pltpu
from jax.experimental.pallas import tpu_sc as plsc

_NUM_ENT = 100000
_DIM = 128
_NUM_REL2 = 400
_BATCH = 16384

_NC = 2
_NS = 16
_NW = _NC * _NS
_BPW = _BATCH // _NW
_CHUNK = 128
_NCHUNK = _BPW // _CHUNK


def _gather_body(emb_hbm, reltab_hbm, sub_hbm, rel_hbm,
                 sub_out, rel_out,
                 sub_idx_v, rel_idx_v, sub_buf, rel_a, rel_b,
                 sem_gs, sem_ws, sem_ga, sem_gb, sem_wa, sem_wb):
    c = lax.axis_index("c")
    s = lax.axis_index("s")
    wid = s * _NC + c
    base = wid * _BPW
    irow = wid * _NCHUNK

    pltpu.sync_copy(sub_hbm.at[pl.ds(irow, _NCHUNK)], sub_idx_v)
    pltpu.sync_copy(rel_hbm.at[pl.ds(irow, _NCHUNK)], rel_idx_v)

    sub_cps = [
        pltpu.async_copy(emb_hbm.at[sub_idx_v.at[j]],
                         sub_buf.at[pl.ds(j * _CHUNK, _CHUNK)], sem_gs)
        for j in range(_NCHUNK)
    ]
    ga = pltpu.async_copy(reltab_hbm.at[rel_idx_v.at[0]], rel_a, sem_ga)

    ga.wait()
    wa = pltpu.async_copy(rel_a, rel_out.at[pl.ds(base, _CHUNK)], sem_wa)
    gb = pltpu.async_copy(reltab_hbm.at[rel_idx_v.at[1]], rel_b, sem_gb)

    for cp in sub_cps:
        cp.wait()
    ws = pltpu.async_copy(sub_buf, sub_out.at[pl.ds(base, _BPW)], sem_ws)

    gb.wait()
    wb = pltpu.async_copy(rel_b, rel_out.at[pl.ds(base + _CHUNK, _CHUNK)],
                          sem_wb)
    wa.wait()
    ga2 = pltpu.async_copy(reltab_hbm.at[rel_idx_v.at[2]], rel_a, sem_ga)
    ga2.wait()
    wa2 = pltpu.async_copy(rel_a,
                           rel_out.at[pl.ds(base + 2 * _CHUNK, _CHUNK)],
                           sem_wa)
    wb.wait()
    gb2 = pltpu.async_copy(reltab_hbm.at[rel_idx_v.at[3]], rel_b, sem_gb)
    gb2.wait()
    wb2 = pltpu.async_copy(rel_b,
                           rel_out.at[pl.ds(base + 3 * _CHUNK, _CHUNK)],
                           sem_wb)

    ws.wait()
    wa2.wait()
    wb2.wait()


@functools.partial(
    pl.kernel,
    out_type=(
        jax.ShapeDtypeStruct((_BATCH, _DIM), jnp.float32),
        jax.ShapeDtypeStruct((_BATCH, _DIM), jnp.float32),
    ),
    mesh=plsc.VectorSubcoreMesh(core_axis_name="c", subcore_axis_name="s"),
    scratch_types=(
        pltpu.VMEM((_NCHUNK, _CHUNK), jnp.int32),
        pltpu.VMEM((_NCHUNK, _CHUNK), jnp.int32),
        pltpu.VMEM((_BPW, _DIM), jnp.float32),
        pltpu.VMEM((_CHUNK, _DIM), jnp.float32),
        pltpu.VMEM((_CHUNK, _DIM), jnp.float32),
        pltpu.SemaphoreType.DMA,
        pltpu.SemaphoreType.DMA,
        pltpu.SemaphoreType.DMA,
        pltpu.SemaphoreType.DMA,
        pltpu.SemaphoreType.DMA,
        pltpu.SemaphoreType.DMA,
    ),
)
def _sc_gathers(emb_hbm, reltab_hbm, sub_hbm, rel_hbm, sub_out, rel_out,
                sub_idx_v, rel_idx_v, sub_buf, rel_a, rel_b,
                sem_gs, sem_ws, sem_ga, sem_gb, sem_wa, sem_wb):
    _gather_body(emb_hbm, reltab_hbm, sub_hbm, rel_hbm, sub_out, rel_out,
                 sub_idx_v, rel_idx_v, sub_buf, rel_a, rel_b,
                 sem_gs, sem_ws, sem_ga, sem_gb, sem_wa, sem_wb)


_COPY_ROWS = 5000


def _copy_body(x_ref, o_ref):
    o_ref[...] = x_ref[...]


_tc_copy = pl.pallas_call(
    _copy_body,
    out_shape=jax.ShapeDtypeStruct((_NUM_ENT, _DIM), jnp.float32),
    grid=(_NUM_ENT // _COPY_ROWS,),
    in_specs=[pl.BlockSpec((_COPY_ROWS, _DIM), lambda i: (i, 0))],
    out_specs=pl.BlockSpec((_COPY_ROWS, _DIM), lambda i: (i, 0)),
)


def kernel(init_embed, init_rel, edge_index, edge_type, sub, rel):
    sub2 = sub.astype(jnp.int32).reshape(_BATCH // _CHUNK, _CHUNK)
    rel2 = rel.astype(jnp.int32).reshape(_BATCH // _CHUNK, _CHUNK)
    sub_emb, rel_emb = _sc_gathers(init_embed, init_rel, sub2, rel2)
    x_out = _tc_copy(init_embed)
    return (sub_emb, rel_emb, x_out)
